# baseline (speedup 1.0000x reference)
import jax
import jax.numpy as jnp
from jax import lax
from jax.experimental import pallas as pl
from jax.experimental.pallas import tpu as pltpu
from jax.experimental.pallas import tpu_sc as plsc

NUM_NODES = 10000
NUM_EDGES = 320000
D = 128
NC = 2
NS = 16
NW = NC * NS
HC = 80
CPT = NUM_EDGES // NW // HC
UPT = 2 * CPT
KB_ = 2
NBODY = UPT // KB_
SWU = 10
IDXR = 2 * SWU
SWB = SWU // KB_
NSL = NUM_NODES // HC
HCH = 80
CPH = NUM_EDGES // NS // HCH

_LANES = 16
_mesh = plsc.VectorSubcoreMesh(core_axis_name="c", subcore_axis_name="s")


def _fill_f32(ref, n, value):
    v = jnp.full((_LANES,), value, jnp.float32)
    for k in range(n // _LANES):
        ref[pl.ds(k * _LANES, _LANES)] = v


def _hist_body(ep_hbm, deg0_hbm, deg1_hbm, idx2, vbuf, hist, sem):
    c = lax.axis_index("c")
    s = lax.axis_index("s")
    _fill_f32(vbuf, HCH, 0.0)
    for t in range(8):
        j = t * NS + s

        @pl.when(j < NUM_NODES // HCH)
        def _():
            pltpu.sync_copy(vbuf, hist.at[pl.ds(j * HCH, HCH)])

    plsc.subcore_barrier()
    _fill_f32(vbuf, HCH, 1.0)
    row0 = (c * NS + s) * CPH
    pltpu.sync_copy(ep_hbm.at[pl.ds(row0, CPH)], idx2)

    def fire(j, carry):
        pltpu.async_copy(vbuf, hist.at[idx2.at[j, 0]], sem, add=True)
        return carry

    lax.fori_loop(0, CPH, fire, 0)

    def drain(j, carry):
        pltpu.make_async_copy(vbuf, hist.at[idx2.at[j, 0]], sem).wait()
        return carry

    lax.fori_loop(0, CPH, drain, 0)
    plsc.subcore_barrier()

    @pl.when(jnp.logical_and(s == 0, c == 0))
    def _():
        pltpu.sync_copy(hist, deg0_hbm)

    @pl.when(jnp.logical_and(s == 0, c == 1))
    def _():
        pltpu.sync_copy(hist, deg1_hbm)


def _prop_body(y_hbm, g_hbm, s_hbm, p_hbm, gbuf, sbuf, r0, r1,
               acc, sem_i, sem_z,
               gs0, gs1, ss0, ss1):
    c = lax.axis_index("c")
    s = lax.axis_index("s")
    wid = c * NS + s
    base = wid * UPT
    bufs = (r0, r1)
    gsem = (gs0, gs1)
    ssem = (ss0, ss1)

    pltpu.sync_copy(g_hbm.at[pl.ds(base, SWU)], gbuf.at[pl.ds(0, SWU)])
    pltpu.sync_copy(s_hbm.at[pl.ds(base, SWU)], sbuf.at[pl.ds(0, SWU)])
    pltpu.async_copy(g_hbm.at[pl.ds(base + SWU, SWU)],
                     gbuf.at[pl.ds(SWU, SWU)], sem_i)
    pltpu.async_copy(s_hbm.at[pl.ds(base + SWU, SWU)],
                     sbuf.at[pl.ds(SWU, SWU)], sem_i)

    def zrow(i, carry):
        for k in range(D // _LANES):
            r0[i, pl.ds(k * _LANES, _LANES)] = jnp.zeros((_LANES,),
                                                         jnp.float32)
        return carry

    lax.fori_loop(0, HC, zrow, 0)
    for t in range(8):
        j = t * NS + s

        @pl.when(j < NSL)
        def _():
            pltpu.async_copy(r0, acc.at[pl.ds(j * HC, HC)], sem_z)

    for t in range(8):
        j = t * NS + s

        @pl.when(j < NSL)
        def _():
            pltpu.make_async_copy(r0, acc.at[pl.ds(j * HC, HC)],
                                  sem_z).wait()

    plsc.subcore_barrier()

    def body(g, carry):
        u0 = g * KB_
        for i in range(KB_):
            u = u0 - KB_ + i
            ur = lax.rem(u + UPT, IDXR)

            @pl.when(g > 0)
            def _():
                pltpu.make_async_copy(bufs[i], acc.at[sbuf.at[ur, 0]],
                                      ssem[i]).wait()

        at_switch = jnp.logical_and(lax.rem(g, SWB) == 0, g > 0)

        @pl.when(at_switch)
        def _():
            pltpu.make_async_copy(
                g_hbm.at[pl.ds(base + u0, SWU)],
                gbuf.at[pl.ds(lax.rem(u0, IDXR), SWU)], sem_i).wait()
            pltpu.make_async_copy(
                s_hbm.at[pl.ds(base + u0, SWU)],
                sbuf.at[pl.ds(lax.rem(u0, IDXR), SWU)], sem_i).wait()

        @pl.when(jnp.logical_and(at_switch, g < NBODY - SWB))
        def _():
            nxt = u0 + SWU
            pltpu.async_copy(g_hbm.at[pl.ds(base + nxt, SWU)],
                             gbuf.at[pl.ds(lax.rem(nxt, IDXR), SWU)],
                             sem_i)
            pltpu.async_copy(s_hbm.at[pl.ds(base + nxt, SWU)],
                             sbuf.at[pl.ds(lax.rem(nxt, IDXR), SWU)],
                             sem_i)

        for i in range(KB_):
            u = u0 + i
            pltpu.async_copy(y_hbm.at[gbuf.at[lax.rem(u, IDXR), 0]],
                             bufs[i], gsem[i])
        for i in range(KB_):
            u = u0 + i
            pltpu.make_async_copy(y_hbm.at[gbuf.at[lax.rem(u, IDXR), 0]],
                                  bufs[i], gsem[i]).wait()
            pltpu.async_copy(bufs[i],
                             acc.at[sbuf.at[lax.rem(u, IDXR), 0]],
                             ssem[i], add=True)
        return carry

    lax.fori_loop(0, NBODY, body, 0)
    for i in range(KB_):
        u = (NBODY - 1) * KB_ + i
        pltpu.make_async_copy(bufs[i],
                              acc.at[sbuf.at[lax.rem(u, IDXR), 0]],
                              ssem[i]).wait()
    plsc.subcore_barrier()
    for t in range(8):
        j = t * NS + s

        @pl.when(j < NSL)
        def _():
            pltpu.async_copy(acc.at[pl.ds(j * HC, HC)],
                             p_hbm.at[c, pl.ds(j * HC, HC)], sem_z)

    for t in range(8):
        j = t * NS + s

        @pl.when(j < NSL)
        def _():
            pltpu.make_async_copy(acc.at[pl.ds(j * HC, HC)],
                                  p_hbm.at[c, pl.ds(j * HC, HC)],
                                  sem_z).wait()


def _scale_body(d0_ref, d1_ref, x_ref, y_ref):
    d = d0_ref[...] + d1_ref[...]
    r = lax.rsqrt(jnp.maximum(d, 1.0))
    y_ref[...] = r * x_ref[...]


def _final_body(d0_ref, d1_ref, p_ref, o_ref):
    d = d0_ref[...] + d1_ref[...]
    r = lax.rsqrt(jnp.maximum(d, 1.0))
    o_ref[...] = r * (p_ref[0] + p_ref[1])


_hist = pl.kernel(
    _hist_body,
    out_type=(
        jax.ShapeDtypeStruct((NUM_NODES,), jnp.float32),
        jax.ShapeDtypeStruct((NUM_NODES,), jnp.float32),
    ),
    mesh=_mesh,
    scratch_types=[
        pltpu.VMEM((CPH, 1, HCH), jnp.int32),
        pltpu.VMEM((HCH,), jnp.float32),
        pltpu.VMEM_SHARED((NUM_NODES,), jnp.float32),
        pltpu.SemaphoreType.DMA,
    ],
)

_prop = pl.kernel(
    _prop_body,
    out_type=jax.ShapeDtypeStruct((NC, NUM_NODES, D), jnp.float32),
    mesh=_mesh,
    scratch_types=(
        [pltpu.VMEM((IDXR, 1, HC), jnp.int32)] * 2
        + [pltpu.VMEM((HC, D), jnp.float32)] * KB_
        + [pltpu.VMEM_SHARED((NUM_NODES, D), jnp.float32)]
        + [pltpu.SemaphoreType.DMA] * (2 + 2 * KB_)
    ),
)

_scale = pl.pallas_call(
    _scale_body,
    out_shape=jax.ShapeDtypeStruct((NUM_NODES, D), jnp.float32),
)

_final = pl.pallas_call(
    _final_body,
    out_shape=jax.ShapeDtypeStruct((NUM_NODES, D), jnp.float32),
)


@jax.jit
def kernel(x, edge_index):
    ei = edge_index.astype(jnp.int32)
    src3 = ei[0].reshape(NW, CPT, HC)
    dst3 = ei[1].reshape(NW, CPT, HC)
    garr = jnp.concatenate([dst3, src3], axis=1).reshape(NW * UPT, 1, HC)
    sarr = jnp.concatenate([src3, dst3], axis=1).reshape(NW * UPT, 1, HC)
    ep2 = ei.reshape(2 * NUM_EDGES // HCH, 1, HCH)
    deg0, deg1 = _hist(ep2)
    deg0 = deg0.reshape(NUM_NODES, 1)
    deg1 = deg1.reshape(NUM_NODES, 1)
    y = _scale(deg0, deg1, x)
    p = _prop(y, garr, sarr)
    return _final(deg0, deg1, p)


# speedup vs baseline: 1.0221x; 1.0221x over previous
import jax
import jax.numpy as jnp
from jax import lax
from jax.experimental import pallas as pl
from jax.experimental.pallas import tpu as pltpu
from jax.experimental.pallas import tpu_sc as plsc

NUM_NODES = 10000
NUM_EDGES = 320000
D = 128
NC = 2
NS = 16
NW = NC * NS
HC = 80
CPT = NUM_EDGES // NW // HC
UPT = 2 * CPT
KB_ = 2
NBODY = UPT // KB_
SWU = 10
IDXR = 2 * SWU
SWB = SWU // KB_
NSL = NUM_NODES // HC
HCH = 80
CPH = NUM_EDGES // NS // HCH

_LANES = 16
_mesh = plsc.VectorSubcoreMesh(core_axis_name="c", subcore_axis_name="s")


def _fill_f32(ref, n, value):
    v = jnp.full((_LANES,), value, jnp.float32)
    for k in range(n // _LANES):
        ref[pl.ds(k * _LANES, _LANES)] = v


def _hist_body(ep_hbm, deg_hbm, idx2, vbuf, hist, sem):
    c = lax.axis_index("c")
    s = lax.axis_index("s")
    _fill_f32(vbuf, HCH, 0.0)
    for t in range(8):
        j = t * NS + s

        @pl.when(j < NUM_NODES // HCH)
        def _():
            pltpu.sync_copy(vbuf, hist.at[pl.ds(j * HCH, HCH)])

    plsc.subcore_barrier()
    _fill_f32(vbuf, HCH, 1.0)
    row0 = (c * NS + s) * CPH
    pltpu.sync_copy(ep_hbm.at[pl.ds(row0, CPH)], idx2)

    def fire(j, carry):
        pltpu.async_copy(vbuf, hist.at[idx2.at[j, 0]], sem, add=True)
        return carry

    lax.fori_loop(0, CPH, fire, 0)

    def drain(j, carry):
        pltpu.make_async_copy(vbuf, hist.at[idx2.at[j, 0]], sem).wait()
        return carry

    lax.fori_loop(0, CPH, drain, 0)
    plsc.subcore_barrier()

    @pl.when(s == 0)
    def _():
        pltpu.sync_copy(hist, deg_hbm.at[c])


def _prop_body(y_hbm, g_hbm, s_hbm, p_hbm, gbuf, sbuf, r0, r1,
               acc, sem_i, sem_z,
               gs0, gs1, ss0, ss1):
    c = lax.axis_index("c")
    s = lax.axis_index("s")
    wid = c * NS + s
    base = wid * UPT
    bufs = (r0, r1)
    gsem = (gs0, gs1)
    ssem = (ss0, ss1)

    pltpu.sync_copy(g_hbm.at[pl.ds(base, SWU)], gbuf.at[pl.ds(0, SWU)])
    pltpu.sync_copy(s_hbm.at[pl.ds(base, SWU)], sbuf.at[pl.ds(0, SWU)])
    pltpu.async_copy(g_hbm.at[pl.ds(base + SWU, SWU)],
                     gbuf.at[pl.ds(SWU, SWU)], sem_i)
    pltpu.async_copy(s_hbm.at[pl.ds(base + SWU, SWU)],
                     sbuf.at[pl.ds(SWU, SWU)], sem_i)

    def zrow(i, carry):
        for k in range(D // _LANES):
            r0[i, pl.ds(k * _LANES, _LANES)] = jnp.zeros((_LANES,),
                                                         jnp.float32)
        return carry

    lax.fori_loop(0, HC, zrow, 0)
    for t in range(8):
        j = t * NS + s

        @pl.when(j < NSL)
        def _():
            pltpu.async_copy(r0, acc.at[pl.ds(j * HC, HC)], sem_z)

    for t in range(8):
        j = t * NS + s

        @pl.when(j < NSL)
        def _():
            pltpu.make_async_copy(r0, acc.at[pl.ds(j * HC, HC)],
                                  sem_z).wait()

    plsc.subcore_barrier()

    def body(g, carry):
        u0 = g * KB_
        for i in range(KB_):
            u = u0 - KB_ + i
            ur = lax.rem(u + UPT, IDXR)

            @pl.when(g > 0)
            def _():
                pltpu.make_async_copy(bufs[i], acc.at[sbuf.at[ur, 0]],
                                      ssem[i]).wait()

        at_switch = jnp.logical_and(lax.rem(g, SWB) == 0, g > 0)

        @pl.when(at_switch)
        def _():
            pltpu.make_async_copy(
                g_hbm.at[pl.ds(base + u0, SWU)],
                gbuf.at[pl.ds(lax.rem(u0, IDXR), SWU)], sem_i).wait()
            pltpu.make_async_copy(
                s_hbm.at[pl.ds(base + u0, SWU)],
                sbuf.at[pl.ds(lax.rem(u0, IDXR), SWU)], sem_i).wait()

        @pl.when(jnp.logical_and(at_switch, g < NBODY - SWB))
        def _():
            nxt = u0 + SWU
            pltpu.async_copy(g_hbm.at[pl.ds(base + nxt, SWU)],
                             gbuf.at[pl.ds(lax.rem(nxt, IDXR), SWU)],
                             sem_i)
            pltpu.async_copy(s_hbm.at[pl.ds(base + nxt, SWU)],
                             sbuf.at[pl.ds(lax.rem(nxt, IDXR), SWU)],
                             sem_i)

        for i in range(KB_):
            u = u0 + i
            pltpu.async_copy(y_hbm.at[gbuf.at[lax.rem(u, IDXR), 0]],
                             bufs[i], gsem[i])
        for i in range(KB_):
            u = u0 + i
            pltpu.make_async_copy(y_hbm.at[gbuf.at[lax.rem(u, IDXR), 0]],
                                  bufs[i], gsem[i]).wait()
            pltpu.async_copy(bufs[i],
                             acc.at[sbuf.at[lax.rem(u, IDXR), 0]],
                             ssem[i], add=True)
        return carry

    lax.fori_loop(0, NBODY, body, 0)
    for i in range(KB_):
        u = (NBODY - 1) * KB_ + i
        pltpu.make_async_copy(bufs[i],
                              acc.at[sbuf.at[lax.rem(u, IDXR), 0]],
                              ssem[i]).wait()
    plsc.subcore_barrier()
    for t in range(8):
        j = t * NS + s

        @pl.when(j < NSL)
        def _():
            pltpu.async_copy(acc.at[pl.ds(j * HC, HC)],
                             p_hbm.at[c, pl.ds(j * HC, HC)], sem_z)

    for t in range(8):
        j = t * NS + s

        @pl.when(j < NSL)
        def _():
            pltpu.make_async_copy(acc.at[pl.ds(j * HC, HC)],
                                  p_hbm.at[c, pl.ds(j * HC, HC)],
                                  sem_z).wait()


def _scale_body(degt_ref, x_ref, y_ref):
    d = degt_ref[:, 0:1] + degt_ref[:, 1:2]
    r = lax.rsqrt(jnp.maximum(d, 1.0))
    y_ref[...] = r * x_ref[...]


def _final_body(degt_ref, p_ref, o_ref):
    d = degt_ref[:, 0:1] + degt_ref[:, 1:2]
    r = lax.rsqrt(jnp.maximum(d, 1.0))
    o_ref[...] = r * (p_ref[0] + p_ref[1])


_hist = pl.kernel(
    _hist_body,
    out_type=jax.ShapeDtypeStruct((NC, NUM_NODES), jnp.float32),
    mesh=_mesh,
    scratch_types=[
        pltpu.VMEM((CPH, 1, HCH), jnp.int32),
        pltpu.VMEM((HCH,), jnp.float32),
        pltpu.VMEM_SHARED((NUM_NODES,), jnp.float32),
        pltpu.SemaphoreType.DMA,
    ],
)

_prop = pl.kernel(
    _prop_body,
    out_type=jax.ShapeDtypeStruct((NC, NUM_NODES, D), jnp.float32),
    mesh=_mesh,
    scratch_types=(
        [pltpu.VMEM((IDXR, 1, HC), jnp.int32)] * 2
        + [pltpu.VMEM((HC, D), jnp.float32)] * KB_
        + [pltpu.VMEM_SHARED((NUM_NODES, D), jnp.float32)]
        + [pltpu.SemaphoreType.DMA] * (2 + 2 * KB_)
    ),
)

_scale = pl.pallas_call(
    _scale_body,
    out_shape=jax.ShapeDtypeStruct((NUM_NODES, D), jnp.float32),
)

_final = pl.pallas_call(
    _final_body,
    out_shape=jax.ShapeDtypeStruct((NUM_NODES, D), jnp.float32),
)


@jax.jit
def kernel(x, edge_index):
    ei = edge_index.astype(jnp.int32)
    src3 = ei[0].reshape(NW, CPT, HC)
    dst3 = ei[1].reshape(NW, CPT, HC)
    garr = jnp.concatenate([dst3, src3], axis=1).reshape(NW * UPT, 1, HC)
    sarr = jnp.concatenate([src3, dst3], axis=1).reshape(NW * UPT, 1, HC)
    ep2 = ei.reshape(2 * NUM_EDGES // HCH, 1, HCH)
    deg_part = _hist(ep2)
    degt = deg_part.T
    y = _scale(degt, x)
    p = _prop(y, garr, sarr)
    return _final(degt, p)


# speedup vs baseline: 1.0269x; 1.0047x over previous
import jax
import jax.numpy as jnp
from jax import lax
from jax.experimental import pallas as pl
from jax.experimental.pallas import tpu as pltpu
from jax.experimental.pallas import tpu_sc as plsc

NUM_NODES = 10000
NUM_EDGES = 320000
D = 128
NC = 2
NS = 16
NW = NC * NS
HC = 80
CPT = NUM_EDGES // NW // HC
UPT = 2 * CPT
KB_ = 2
NBODY = UPT // KB_
SWU = 10
IDXR = 2 * SWU
SWB = SWU // KB_
NSL = NUM_NODES // HC
HCH = 80
CPH = NUM_EDGES // NS // HCH

_LANES = 16
_mesh = plsc.VectorSubcoreMesh(core_axis_name="c", subcore_axis_name="s")


def _fill_f32(ref, n, value):
    v = jnp.full((_LANES,), value, jnp.float32)
    for k in range(n // _LANES):
        ref[pl.ds(k * _LANES, _LANES)] = v


def _hist_body(ep_hbm, deg_hbm, idx2, vbuf, hist, sem):
    c = lax.axis_index("c")
    s = lax.axis_index("s")
    _fill_f32(vbuf, HCH, 0.0)
    for t in range(8):
        j = t * NS + s

        @pl.when(j < NUM_NODES // HCH)
        def _():
            pltpu.sync_copy(vbuf, hist.at[pl.ds(j * HCH, HCH)])

    plsc.subcore_barrier()
    _fill_f32(vbuf, HCH, 1.0)
    row0 = (c * NS + s) * CPH
    pltpu.sync_copy(ep_hbm.at[pl.ds(row0, CPH)], idx2)

    def fire(j, carry):
        pltpu.async_copy(vbuf, hist.at[idx2.at[j, 0]], sem, add=True)
        return carry

    lax.fori_loop(0, CPH, fire, 0)

    def drain(j, carry):
        pltpu.make_async_copy(vbuf, hist.at[idx2.at[j, 0]], sem).wait()
        return carry

    lax.fori_loop(0, CPH, drain, 0)
    plsc.subcore_barrier()

    @pl.when(s == 0)
    def _():
        pltpu.sync_copy(hist, deg_hbm.at[c])


def _prop_body(y_hbm, g_hbm, s_hbm, p_hbm, gbuf, sbuf, r0, r1,
               acc, sem_i, sem_j, sem_z,
               gs0, gs1, ss0, ss1):
    c = lax.axis_index("c")
    s = lax.axis_index("s")
    wid = c * NS + s
    base = wid * UPT
    bufs = (r0, r1)
    gsem = (gs0, gs1)
    ssem = (ss0, ss1)

    pltpu.sync_copy(g_hbm.at[pl.ds(base, SWU)], gbuf.at[pl.ds(0, SWU)])
    pltpu.sync_copy(s_hbm.at[pl.ds(base, SWU)], sbuf.at[pl.ds(0, SWU)])
    pltpu.async_copy(g_hbm.at[pl.ds(base + SWU, SWU)],
                     gbuf.at[pl.ds(SWU, SWU)], sem_i)
    pltpu.async_copy(s_hbm.at[pl.ds(base + SWU, SWU)],
                     sbuf.at[pl.ds(SWU, SWU)], sem_j)

    def zrow(i, carry):
        for k in range(D // _LANES):
            r0[i, pl.ds(k * _LANES, _LANES)] = jnp.zeros((_LANES,),
                                                         jnp.float32)
        return carry

    lax.fori_loop(0, HC, zrow, 0)
    for t in range(8):
        j = t * NS + s

        @pl.when(j < NSL)
        def _():
            pltpu.async_copy(r0, acc.at[pl.ds(j * HC, HC)], sem_z)

    for t in range(8):
        j = t * NS + s

        @pl.when(j < NSL)
        def _():
            pltpu.make_async_copy(r0, acc.at[pl.ds(j * HC, HC)],
                                  sem_z).wait()

    plsc.subcore_barrier()

    def body(g, carry):
        u0 = g * KB_
        for i in range(KB_):
            u = u0 - KB_ + i
            ur = lax.rem(u + UPT, IDXR)

            @pl.when(g > 0)
            def _():
                pltpu.make_async_copy(bufs[i], acc.at[sbuf.at[ur, 0]],
                                      ssem[i]).wait()

        at_switch = jnp.logical_and(lax.rem(g, SWB) == 0, g > 0)

        @pl.when(at_switch)
        def _():
            pltpu.make_async_copy(
                g_hbm.at[pl.ds(base + u0, SWU)],
                gbuf.at[pl.ds(lax.rem(u0, IDXR), SWU)], sem_i).wait()
            pltpu.make_async_copy(
                s_hbm.at[pl.ds(base + u0, SWU)],
                sbuf.at[pl.ds(lax.rem(u0, IDXR), SWU)], sem_j).wait()

        @pl.when(jnp.logical_and(at_switch, g < NBODY - SWB))
        def _():
            nxt = u0 + SWU
            pltpu.async_copy(g_hbm.at[pl.ds(base + nxt, SWU)],
                             gbuf.at[pl.ds(lax.rem(nxt, IDXR), SWU)],
                             sem_i)
            pltpu.async_copy(s_hbm.at[pl.ds(base + nxt, SWU)],
                             sbuf.at[pl.ds(lax.rem(nxt, IDXR), SWU)],
                             sem_j)

        for i in range(KB_):
            u = u0 + i
            pltpu.async_copy(y_hbm.at[gbuf.at[lax.rem(u, IDXR), 0]],
                             bufs[i], gsem[i])
        for i in range(KB_):
            u = u0 + i
            pltpu.make_async_copy(y_hbm.at[gbuf.at[lax.rem(u, IDXR), 0]],
                                  bufs[i], gsem[i]).wait()
            pltpu.async_copy(bufs[i],
                             acc.at[sbuf.at[lax.rem(u, IDXR), 0]],
                             ssem[i], add=True)
        return carry

    lax.fori_loop(0, NBODY, body, 0)
    for i in range(KB_):
        u = (NBODY - 1) * KB_ + i
        pltpu.make_async_copy(bufs[i],
                              acc.at[sbuf.at[lax.rem(u, IDXR), 0]],
                              ssem[i]).wait()
    plsc.subcore_barrier()
    for t in range(8):
        j = t * NS + s

        @pl.when(j < NSL)
        def _():
            pltpu.async_copy(acc.at[pl.ds(j * HC, HC)],
                             p_hbm.at[c, pl.ds(j * HC, HC)], sem_z)

    for t in range(8):
        j = t * NS + s

        @pl.when(j < NSL)
        def _():
            pltpu.make_async_copy(acc.at[pl.ds(j * HC, HC)],
                                  p_hbm.at[c, pl.ds(j * HC, HC)],
                                  sem_z).wait()


def _scale_body(degt_ref, x_ref, y_ref):
    d = degt_ref[:, 0:1] + degt_ref[:, 1:2]
    r = lax.rsqrt(jnp.maximum(d, 1.0))
    y_ref[...] = r * x_ref[...]


def _final_body(degt_ref, p_ref, o_ref):
    d = degt_ref[:, 0:1] + degt_ref[:, 1:2]
    r = lax.rsqrt(jnp.maximum(d, 1.0))
    o_ref[...] = r * (p_ref[0] + p_ref[1])


_hist = pl.kernel(
    _hist_body,
    out_type=jax.ShapeDtypeStruct((NC, NUM_NODES), jnp.float32),
    mesh=_mesh,
    scratch_types=[
        pltpu.VMEM((CPH, 1, HCH), jnp.int32),
        pltpu.VMEM((HCH,), jnp.float32),
        pltpu.VMEM_SHARED((NUM_NODES,), jnp.float32),
        pltpu.SemaphoreType.DMA,
    ],
)

_prop = pl.kernel(
    _prop_body,
    out_type=jax.ShapeDtypeStruct((NC, NUM_NODES, D), jnp.float32),
    mesh=_mesh,
    scratch_types=(
        [pltpu.VMEM((IDXR, 1, HC), jnp.int32)] * 2
        + [pltpu.VMEM((HC, D), jnp.float32)] * KB_
        + [pltpu.VMEM_SHARED((NUM_NODES, D), jnp.float32)]
        + [pltpu.SemaphoreType.DMA] * (3 + 2 * KB_)
    ),
)

_scale = pl.pallas_call(
    _scale_body,
    out_shape=jax.ShapeDtypeStruct((NUM_NODES, D), jnp.float32),
)

_final = pl.pallas_call(
    _final_body,
    out_shape=jax.ShapeDtypeStruct((NUM_NODES, D), jnp.float32),
)


@jax.jit
def kernel(x, edge_index):
    ei = edge_index.astype(jnp.int32)
    src3 = ei[0].reshape(NW, CPT, HC)
    dst3 = ei[1].reshape(NW, CPT, HC)
    garr = jnp.concatenate([dst3, src3], axis=1).reshape(NW * UPT, 1, HC)
    sarr = jnp.concatenate([src3, dst3], axis=1).reshape(NW * UPT, 1, HC)
    ep2 = ei.reshape(2 * NUM_EDGES // HCH, 1, HCH)
    deg_part = _hist(ep2)
    degt = deg_part.T
    y = _scale(degt, x)
    p = _prop(y, garr, sarr)
    return _final(degt, p)
